# Initial kernel scaffold; baseline (speedup 1.0000x reference)
import jax
import jax.numpy as jnp
from jax.experimental import pallas as pl


def kernel(x, edge_index, edge_weight, W1, b1, W2, b2):
    raise NotImplementedError("write your pallas kernel here")


# speedup vs baseline: 2.8309x; 2.8309x over previous
import functools

import jax
import jax.numpy as jnp
from jax.experimental import pallas as pl

N_NODES = 10000
FEAT = 256
HID = 512
OUT = 256

ROW_BLK = 1000


def _fused_mlp_body(g1_ref, y1_ref, dis_ref, w1_ref, b1_ref, w2_ref, y2_ref):
    dis = dis_ref[...]
    s1 = dis * (g1_ref[...] + y1_ref[...])
    h = jnp.maximum(
        jnp.dot(s1, w1_ref[...], preferred_element_type=jnp.float32)
        + b1_ref[...],
        0.0,
    )
    z = jnp.dot(h, w2_ref[...], preferred_element_type=jnp.float32)
    y2_ref[...] = dis * z


def _fused_mlp(g1, y1, dis2d, W1, b1, W2):
    grid = (N_NODES // ROW_BLK,)
    return pl.pallas_call(
        _fused_mlp_body,
        grid=grid,
        in_specs=[
            pl.BlockSpec((ROW_BLK, FEAT), lambda i: (i, 0)),
            pl.BlockSpec((ROW_BLK, FEAT), lambda i: (i, 0)),
            pl.BlockSpec((ROW_BLK, 1), lambda i: (i, 0)),
            pl.BlockSpec((FEAT, HID), lambda i: (0, 0)),
            pl.BlockSpec((1, HID), lambda i: (0, 0)),
            pl.BlockSpec((HID, OUT), lambda i: (0, 0)),
        ],
        out_specs=pl.BlockSpec((ROW_BLK, OUT), lambda i: (i, 0)),
        out_shape=jax.ShapeDtypeStruct((N_NODES, OUT), jnp.float32),
    )(g1, y1, dis2d, W1, b1, W2)


def _prescale_body(x_ref, dis_ref, y_ref):
    y_ref[...] = dis_ref[...] * x_ref[...]


def _prescale(x, dis2d):
    grid = (N_NODES // ROW_BLK,)
    return pl.pallas_call(
        _prescale_body,
        grid=grid,
        in_specs=[
            pl.BlockSpec((ROW_BLK, FEAT), lambda i: (i, 0)),
            pl.BlockSpec((ROW_BLK, 1), lambda i: (i, 0)),
        ],
        out_specs=pl.BlockSpec((ROW_BLK, FEAT), lambda i: (i, 0)),
        out_shape=jax.ShapeDtypeStruct((N_NODES, FEAT), jnp.float32),
    )(x, dis2d)


def _final_body(g2_ref, y2_ref, dis_ref, b2_ref, out_ref):
    out_ref[...] = dis_ref[...] * (g2_ref[...] + y2_ref[...]) + b2_ref[...]


def _final(g2, y2, dis2d, b2):
    grid = (N_NODES // ROW_BLK,)
    return pl.pallas_call(
        _final_body,
        grid=grid,
        in_specs=[
            pl.BlockSpec((ROW_BLK, OUT), lambda i: (i, 0)),
            pl.BlockSpec((ROW_BLK, OUT), lambda i: (i, 0)),
            pl.BlockSpec((ROW_BLK, 1), lambda i: (i, 0)),
            pl.BlockSpec((1, OUT), lambda i: (0, 0)),
        ],
        out_specs=pl.BlockSpec((ROW_BLK, OUT), lambda i: (i, 0)),
        out_shape=jax.ShapeDtypeStruct((N_NODES, OUT), jnp.float32),
    )(g2, y2, dis2d, b2)


def _aggregate(y, src, dst, ew):
    msg = y[src] * ew[:, None]
    return jnp.zeros((N_NODES, y.shape[1]), jnp.float32).at[dst].add(msg)


def kernel(x, edge_index, edge_weight, W1, b1, W2, b2):
    src = edge_index[0].astype(jnp.int32)
    dst = edge_index[1].astype(jnp.int32)
    ew = edge_weight.astype(jnp.float32)

    deg = jnp.zeros((N_NODES,), jnp.float32).at[dst].add(ew) + 1.0
    dis = jnp.where(deg > 0.0, jax.lax.rsqrt(deg), 0.0)
    dis2d = dis[:, None]

    y1 = _prescale(x, dis2d)
    g1 = _aggregate(y1, src, dst, ew)
    y2 = _fused_mlp(g1, y1, dis2d, W1, b1.reshape(1, HID), W2)
    g2 = _aggregate(y2, src, dst, ew)
    return _final(g2, y2, dis2d, b2.reshape(1, OUT))
